# baseline (speedup 1.0000x reference)
import functools

import jax
import jax.numpy as jnp
from jax import lax
from jax.experimental import pallas as pl
from jax.experimental.pallas import tpu as pltpu
from jax.experimental.pallas import tpu_sc as plsc

B = 4096
DIM_NUM = 26
DIM_EMB = 128
MAX_LEN = 50
DIM_BIAS = DIM_NUM + MAX_LEN
N_DENSE = DIM_NUM + 1
N_OUT = N_DENSE + MAX_LEN
LANES = 16
NCHUNK = DIM_EMB // LANES

_NC = 2
_NS = 16
NW = _NC * _NS
BPW = B // NW


def _body(xn_hbm, xc_hbm, tab_hbm, w_hbm, b_hbm, out_hbm,
          xn_v, xc_v, w_v, b_v, row0_v, row1_v, gs0, gs1, os0, os1):
    cid = lax.axis_index("c")
    sid = lax.axis_index("s")
    wid = sid * _NC + cid
    base = wid * BPW

    pltpu.sync_copy(xn_hbm.at[pl.ds(base, BPW)], xn_v)
    pltpu.sync_copy(xc_hbm.at[pl.ds(base, BPW)], xc_v)
    pltpu.sync_copy(w_hbm, w_v)
    pltpu.sync_copy(b_hbm, b_v)

    def issue_gather(i, buf, sem):
        pltpu.async_copy(tab_hbm.at[xc_v.at[i]], buf.at[pl.ds(N_DENSE, MAX_LEN)], sem)

    def wait_gather(buf, sem):
        pltpu.make_async_copy(
            tab_hbm.at[xc_v.at[0]], buf.at[pl.ds(N_DENSE, MAX_LEN)], sem
        ).wait()

    def issue_out(i, buf, sem):
        pltpu.async_copy(buf, out_hbm.at[base + i], sem)

    def wait_out(buf, sem):
        pltpu.make_async_copy(buf, out_hbm.at[base], sem).wait()

    for c in range(NCHUNK):
        sl = pl.ds(c * LANES, LANES)
        v = w_v[0, sl]
        row0_v[0, sl] = v
        row1_v[0, sl] = v

    def compute(i, buf):
        xs0 = xn_v[i, pl.ds(0, LANES)]
        xs1 = xn_v[i, pl.ds(LANES, LANES)]
        for j in range(1, N_DENSE):
            col = j - 1
            xs = xs0[col] if col < LANES else xs1[col - LANES]
            for c in range(NCHUNK):
                sl = pl.ds(c * LANES, LANES)
                buf[j, sl] = w_v[j, sl] * xs + b_v[col, sl]
        for l in range(MAX_LEN):
            for c in range(NCHUNK):
                sl = pl.ds(c * LANES, LANES)
                plsc.addupdate(buf.at[N_DENSE + l, sl], b_v[DIM_NUM + l, sl])

    issue_gather(0, row0_v, gs0)
    issue_gather(1, row1_v, gs1)

    def pair(k, carry):
        i0 = 2 * k
        i1 = 2 * k + 1
        wait_gather(row0_v, gs0)
        compute(i0, row0_v)
        issue_out(i0, row0_v, os0)
        wait_gather(row1_v, gs1)
        compute(i1, row1_v)
        issue_out(i1, row1_v, os1)
        nx0 = jnp.minimum(i0 + 2, BPW - 1)
        nx1 = jnp.minimum(i1 + 2, BPW - 1)
        wait_out(row0_v, os0)
        issue_gather(nx0, row0_v, gs0)
        wait_out(row1_v, os1)
        issue_gather(nx1, row1_v, gs1)
        return 0

    lax.fori_loop(0, BPW // 2, pair, 0)
    wait_gather(row0_v, gs0)
    wait_gather(row1_v, gs1)


@jax.jit
def kernel(x_num, x_cat, emb_table, weight, bias):
    run = functools.partial(
        pl.kernel,
        mesh=plsc.VectorSubcoreMesh(core_axis_name="c", subcore_axis_name="s"),
        out_type=jax.ShapeDtypeStruct((B, N_OUT, DIM_EMB), jnp.float32),
        scratch_types=[
            pltpu.VMEM((BPW, 2 * LANES), jnp.float32),
            pltpu.VMEM((BPW, MAX_LEN), jnp.int32),
            pltpu.VMEM((N_DENSE, DIM_EMB), jnp.float32),
            pltpu.VMEM((DIM_BIAS, DIM_EMB), jnp.float32),
            pltpu.VMEM((N_OUT, DIM_EMB), jnp.float32),
            pltpu.VMEM((N_OUT, DIM_EMB), jnp.float32),
            pltpu.SemaphoreType.DMA,
            pltpu.SemaphoreType.DMA,
            pltpu.SemaphoreType.DMA,
            pltpu.SemaphoreType.DMA,
        ],
    )(_body)
    x_num_p = jnp.pad(x_num, ((0, 0), (0, 2 * LANES - DIM_NUM)))
    return run(x_num_p, x_cat, emb_table, weight, bias)


# speedup vs baseline: 1.8517x; 1.8517x over previous
import functools

import jax
import jax.numpy as jnp
from jax import lax
from jax.experimental import pallas as pl
from jax.experimental.pallas import tpu as pltpu
from jax.experimental.pallas import tpu_sc as plsc

B = 4096
DIM_NUM = 26
DIM_EMB = 128
MAX_LEN = 50
DIM_BIAS = DIM_NUM + MAX_LEN
N_DENSE = DIM_NUM + 1
N_OUT = N_DENSE + MAX_LEN
LANES = 16
NCHUNK = DIM_EMB // LANES

_NC = 2
_NS = 16
NW = _NC * _NS
BPW = B // NW


def _body(xn_hbm, xc_hbm, tab_hbm, w_hbm, b_hbm, out_hbm,
          xn_v, xc_v, w_v, b_v, row0_v, row1_v, gs0, gs1, os0, os1):
    cid = lax.axis_index("c")
    sid = lax.axis_index("s")
    wid = sid * _NC + cid
    base = wid * BPW

    pltpu.sync_copy(xn_hbm.at[pl.ds(base, BPW)], xn_v)
    pltpu.sync_copy(xc_hbm.at[pl.ds(base, BPW)], xc_v)
    pltpu.sync_copy(w_hbm, w_v)
    pltpu.sync_copy(b_hbm, b_v)

    def issue_gather(i, buf, sem):
        pltpu.async_copy(tab_hbm.at[xc_v.at[i]], buf.at[pl.ds(N_DENSE, MAX_LEN)], sem)

    def wait_gather(buf, sem):
        pltpu.make_async_copy(
            tab_hbm.at[xc_v.at[0]], buf.at[pl.ds(N_DENSE, MAX_LEN)], sem
        ).wait()

    def issue_out(i, buf, sem):
        pltpu.async_copy(buf, out_hbm.at[base + i], sem)

    def wait_out(buf, sem):
        pltpu.make_async_copy(buf, out_hbm.at[base], sem).wait()

    for c in range(NCHUNK):
        sl = pl.ds(c * LANES, LANES)
        v = w_v[0, sl]
        row0_v[0, sl] = v
        row1_v[0, sl] = v

    def compute(i, buf):
        xs0 = xn_v[i, pl.ds(0, LANES)]
        xs1 = xn_v[i, pl.ds(LANES, LANES)]
        for j in range(1, N_DENSE):
            col = j - 1
            xs = xs0[col] if col < LANES else xs1[col - LANES]
            for c in range(NCHUNK):
                sl = pl.ds(c * LANES, LANES)
                buf[j, sl] = w_v[j, sl] * xs + b_v[col, sl]
        def emb(l, _):
            for c in range(NCHUNK):
                sl = pl.ds(c * LANES, LANES)
                plsc.addupdate(buf.at[N_DENSE + l, sl], b_v[DIM_NUM + l, sl])
            return 0

        lax.fori_loop(0, MAX_LEN, emb, 0)

    issue_gather(0, row0_v, gs0)
    issue_gather(1, row1_v, gs1)

    def pair(k, carry):
        i0 = 2 * k
        i1 = 2 * k + 1
        wait_gather(row0_v, gs0)
        compute(i0, row0_v)
        issue_out(i0, row0_v, os0)
        wait_gather(row1_v, gs1)
        compute(i1, row1_v)
        issue_out(i1, row1_v, os1)
        nx0 = jnp.minimum(i0 + 2, BPW - 1)
        nx1 = jnp.minimum(i1 + 2, BPW - 1)
        wait_out(row0_v, os0)
        issue_gather(nx0, row0_v, gs0)
        wait_out(row1_v, os1)
        issue_gather(nx1, row1_v, gs1)
        return 0

    lax.fori_loop(0, BPW // 2, pair, 0)
    wait_gather(row0_v, gs0)
    wait_gather(row1_v, gs1)


@jax.jit
def kernel(x_num, x_cat, emb_table, weight, bias):
    run = functools.partial(
        pl.kernel,
        mesh=plsc.VectorSubcoreMesh(core_axis_name="c", subcore_axis_name="s"),
        out_type=jax.ShapeDtypeStruct((B, N_OUT, DIM_EMB), jnp.float32),
        scratch_types=[
            pltpu.VMEM((BPW, 2 * LANES), jnp.float32),
            pltpu.VMEM((BPW, MAX_LEN), jnp.int32),
            pltpu.VMEM((N_DENSE, DIM_EMB), jnp.float32),
            pltpu.VMEM((DIM_BIAS, DIM_EMB), jnp.float32),
            pltpu.VMEM((N_OUT, DIM_EMB), jnp.float32),
            pltpu.VMEM((N_OUT, DIM_EMB), jnp.float32),
            pltpu.SemaphoreType.DMA,
            pltpu.SemaphoreType.DMA,
            pltpu.SemaphoreType.DMA,
            pltpu.SemaphoreType.DMA,
        ],
    )(_body)
    x_num_p = jnp.pad(x_num, ((0, 0), (0, 2 * LANES - DIM_NUM)))
    return run(x_num_p, x_cat, emb_table, weight, bias)


# speedup vs baseline: 2.2170x; 1.1973x over previous
import functools

import jax
import jax.numpy as jnp
from jax import lax
from jax.experimental import pallas as pl
from jax.experimental.pallas import tpu as pltpu
from jax.experimental.pallas import tpu_sc as plsc

B = 4096
DIM_NUM = 26
DIM_EMB = 128
MAX_LEN = 50
DIM_BIAS = DIM_NUM + MAX_LEN
N_DENSE = DIM_NUM + 1
N_OUT = N_DENSE + MAX_LEN
LANES = 16
NCHUNK = DIM_EMB // LANES

_NC = 2
_NS = 16
NW = _NC * _NS
BPW = B // NW


def _body(xn_hbm, xc_hbm, tab_hbm, w_hbm, b_hbm, out_hbm,
          xn_v, xc_v, w_v, b_v, row0_v, row1_v, gs0, gs1, os0, os1):
    cid = lax.axis_index("c")
    sid = lax.axis_index("s")
    wid = sid * _NC + cid
    base = wid * BPW

    pltpu.sync_copy(xn_hbm.at[pl.ds(base, BPW)], xn_v)
    pltpu.sync_copy(xc_hbm.at[pl.ds(base, BPW)], xc_v)
    pltpu.sync_copy(w_hbm, w_v)
    pltpu.sync_copy(b_hbm, b_v)

    def issue_gather(i, buf, sem):
        pltpu.async_copy(tab_hbm.at[xc_v.at[i]], buf.at[pl.ds(N_DENSE, MAX_LEN)], sem)

    def wait_gather(buf, sem):
        pltpu.make_async_copy(
            tab_hbm.at[xc_v.at[0]], buf.at[pl.ds(N_DENSE, MAX_LEN)], sem
        ).wait()

    def issue_out(i, buf, sem):
        pltpu.async_copy(buf, out_hbm.at[base + i], sem)

    def wait_out(buf, sem):
        pltpu.make_async_copy(buf, out_hbm.at[base], sem).wait()

    for c in range(NCHUNK):
        sl = pl.ds(c * LANES, LANES)
        v = w_v[0, sl]
        row0_v[0, sl] = v
        row1_v[0, sl] = v

    def dense_pair(i0, i1):
        xa0 = xn_v[i0, pl.ds(0, LANES)]
        xa1 = xn_v[i0, pl.ds(LANES, LANES)]
        xb0 = xn_v[i1, pl.ds(0, LANES)]
        xb1 = xn_v[i1, pl.ds(LANES, LANES)]
        for j in range(1, N_DENSE):
            col = j - 1
            xa = xa0[col] if col < LANES else xa1[col - LANES]
            xb = xb0[col] if col < LANES else xb1[col - LANES]
            for c in range(NCHUNK):
                sl = pl.ds(c * LANES, LANES)
                wv = w_v[j, sl]
                bv = b_v[col, sl]
                row0_v[j, sl] = wv * xa + bv
                row1_v[j, sl] = wv * xb + bv

    def emb_bias(buf):
        def emb(l, _):
            for c in range(NCHUNK):
                sl = pl.ds(c * LANES, LANES)
                plsc.addupdate(buf.at[N_DENSE + l, sl], b_v[DIM_NUM + l, sl])
            return 0

        lax.fori_loop(0, MAX_LEN, emb, 0, unroll=5)

    issue_gather(0, row0_v, gs0)
    issue_gather(1, row1_v, gs1)

    def pair(k, carry):
        i0 = 2 * k
        i1 = 2 * k + 1
        dense_pair(i0, i1)
        wait_gather(row0_v, gs0)
        emb_bias(row0_v)
        issue_out(i0, row0_v, os0)
        wait_gather(row1_v, gs1)
        emb_bias(row1_v)
        issue_out(i1, row1_v, os1)
        nx0 = jnp.minimum(i0 + 2, BPW - 1)
        nx1 = jnp.minimum(i1 + 2, BPW - 1)
        wait_out(row0_v, os0)
        issue_gather(nx0, row0_v, gs0)
        wait_out(row1_v, os1)
        issue_gather(nx1, row1_v, gs1)
        return 0

    lax.fori_loop(0, BPW // 2, pair, 0)
    wait_gather(row0_v, gs0)
    wait_gather(row1_v, gs1)


@jax.jit
def kernel(x_num, x_cat, emb_table, weight, bias):
    run = functools.partial(
        pl.kernel,
        mesh=plsc.VectorSubcoreMesh(core_axis_name="c", subcore_axis_name="s"),
        out_type=jax.ShapeDtypeStruct((B, N_OUT, DIM_EMB), jnp.float32),
        scratch_types=[
            pltpu.VMEM((BPW, 2 * LANES), jnp.float32),
            pltpu.VMEM((BPW, MAX_LEN), jnp.int32),
            pltpu.VMEM((N_DENSE, DIM_EMB), jnp.float32),
            pltpu.VMEM((DIM_BIAS, DIM_EMB), jnp.float32),
            pltpu.VMEM((N_OUT, DIM_EMB), jnp.float32),
            pltpu.VMEM((N_OUT, DIM_EMB), jnp.float32),
            pltpu.SemaphoreType.DMA,
            pltpu.SemaphoreType.DMA,
            pltpu.SemaphoreType.DMA,
            pltpu.SemaphoreType.DMA,
        ],
    )(_body)
    x_num_p = jnp.pad(x_num, ((0, 0), (0, 2 * LANES - DIM_NUM)))
    return run(x_num_p, x_cat, emb_table, weight, bias)
